# baseline (speedup 1.0000x reference)
import jax
import jax.numpy as jnp
from jax import lax
from jax.experimental import pallas as pl
from jax.experimental.pallas import tpu as pltpu
from jax.experimental.pallas import tpu_sc as plsc

B = 16384
N = 1000
K = 100
EMB = 512
PENALTY = 10.0
NPAD = 1008
OUTW = 128
NBUF = 4
NBKT = 32
CAND_CAP = 160
CAND_BUF = 1024
NC = 2
NS = 16
NW = NC * NS
ROWS_PER_W = B // NW
TC_ROWS = 256


def _iota16():
    return lax.broadcasted_iota(jnp.int32, (16,), 0)


class _Ops:
    def __init__(self, vsort1, rev, minimum, maximum, where, le):
        self.vsort1_f = vsort1
        self.rev_f = rev
        self.minimum = minimum
        self.maximum = maximum
        self.where = where
        self.le = le

    def vsort1(self, pair):
        if pair is None:
            return None
        return self.vsort1_f(pair)

    def rev(self, pair):
        if pair is None:
            return None
        return (self.rev_f(pair[0]), self.rev_f(pair[1]))

    def minmax(self, a, b):
        if a is None and b is None:
            return None, None
        if b is None:
            return a, None
        if a is None:
            return b, None
        ka, pa = a
        kb, pb = b
        m = self.le(ka, kb)
        lo = (self.minimum(ka, kb), self.where(m, pa, pb))
        hi = (self.maximum(ka, kb), self.where(m, pb, pa))
        return lo, hi

    def min_only(self, a, b):
        if b is None:
            return a
        if a is None:
            return b
        ka, pa = a
        kb, pb = b
        m = self.le(ka, kb)
        return (self.minimum(ka, kb), self.where(m, pa, pb))


def _bitonic(Z, ops, keep=None):
    n = len(Z)
    if keep is not None and keep <= 0:
        return [None] * n
    if n == 1:
        return [ops.vsort1(Z[0])]
    half = n // 2
    prune_hi = keep is not None and keep <= half
    lo, hi = [], []
    for i in range(half):
        if prune_hi:
            lo.append(ops.min_only(Z[i], Z[i + half]))
            hi.append(None)
        else:
            l, h = ops.minmax(Z[i], Z[i + half])
            lo.append(l)
            hi.append(h)
    khi = None if keep is None else keep - half
    return _bitonic(lo, ops, keep) + _bitonic(hi, ops, khi)


def _merge(A, Bl, ops, keep=None):
    n = 1
    while n < max(len(A), len(Bl)):
        n *= 2
    A = A + [None] * (n - len(A))
    Bl = Bl + [None] * (n - len(Bl))
    Z = A + [ops.rev(x) for x in reversed(Bl)]
    return _bitonic(Z, ops, keep)


def _sort_network(pairs, ops, keep=None):
    if len(pairs) == 1:
        return [ops.vsort1(pairs[0])]
    h = len(pairs) // 2
    return _merge(_sort_network(pairs[:h], ops),
                  _sort_network(pairs[h:], ops), ops, keep)


def _jnp_ops():
    def vsort1(pair):
        k, p = pair
        ks, ps = plsc.sort_key_val(k, p)
        return (ks, ps)
    return _Ops(vsort1, lambda v: lax.rev(v, (0,)), jnp.minimum,
                jnp.maximum, jnp.where, lambda a, b: a <= b)


def _topk_kernel(dist_hbm, theta_hbm, sd_hbm, th_hbm, ix_hbm, *scr):
    dbufs = scr[0:NBUF]
    tbufs = scr[NBUF:2 * NBUF]
    sdbufs = scr[2 * NBUF:3 * NBUF]
    thbufs = scr[3 * NBUF:4 * NBUF]
    ixbufs = scr[4 * NBUF:5 * NBUF]
    cand = scr[5 * NBUF]
    hist = scr[5 * NBUF + 1]
    sem_in = scr[5 * NBUF + 2:5 * NBUF + 2 + NBUF]
    sem_out = scr[5 * NBUF + 2 + NBUF:5 * NBUF + 2 + 2 * NBUF]

    wid = lax.axis_index("s") * NC + lax.axis_index("c")
    base = wid * ROWS_PER_W
    iota = _iota16()
    ones16 = jnp.full((16,), 1, jnp.int32)
    infv = jnp.full((16,), jnp.inf, jnp.float32)
    zf = jnp.zeros((16,), jnp.float32)
    zi = jnp.zeros((16,), jnp.int32)

    def in_copies(b, r):
        return (pltpu.make_async_copy(dist_hbm.at[base + r],
                                      dbufs[b].at[pl.ds(0, N)], sem_in[b]),
                pltpu.make_async_copy(theta_hbm.at[base + r],
                                      tbufs[b], sem_in[b]))

    def out_copies(b, r):
        return (pltpu.make_async_copy(sdbufs[b], sd_hbm.at[base + r], sem_out[b]),
                pltpu.make_async_copy(thbufs[b], th_hbm.at[base + r], sem_out[b]),
                pltpu.make_async_copy(ixbufs[b], ix_hbm.at[base + r], sem_out[b]))

    for b in range(NBUF):
        dbufs[b][pl.ds(992, 16)] = infv
        sdbufs[b][pl.ds(112, 16)] = zf
        thbufs[b][pl.ds(112, 16)] = zf
        ixbufs[b][pl.ds(112, 16)] = zi
        for c in in_copies(b, b):
            c.start()

    lane15 = jnp.full((16,), 15, jnp.int32)
    NV = CAND_CAP // 16

    def prefill(cb):
        for j in range(NV):
            cand[pl.ds(cb + j * 16, 16)] = jnp.full((16,), NPAD - 1, jnp.int32)

    _shift_idx = [jnp.maximum(iota - (1 << s), 0) for s in range(4)]

    def c_one(dbuf, c, off_v, cbase, thresh):
        v = dbuf[pl.ds(c * 16, 16)]
        sel = v < thresh
        pfx = jnp.where(sel, 1, 0)
        for s in range(4):
            t = jnp.take(pfx, _shift_idx[s])
            pfx = pfx + jnp.where(iota >= (1 << s), t, 0)
        pos = pfx - 1 + off_v
        if cbase:
            pos = pos + cbase
        plsc.store_scatter(cand, [pos], c * 16 + iota, mask=sel)
        return off_v + jnp.take(pfx, lane15)

    def splice(cb, oa, ob):
        basev = oa + iota + cb
        for j in range(NV):
            iv = cand[pl.ds(cb + 512 + j * 16, 16)]
            ivm = jnp.where(j * 16 + iota < ob, iv, NPAD - 1)
            plsc.store_scatter(cand, [basev + j * 16], ivm)

    def compact1(dbuf, cb, thresh):
        def body(i, carry):
            oa, ob = carry
            for u in range(3):
                oa = c_one(dbuf, i * 3 + u, oa, cb, thresh)
                ob = c_one(dbuf, 31 + i * 3 + u, ob, cb + 512, thresh)
            return oa, ob
        z = jnp.zeros((16,), jnp.int32)
        oa, ob = lax.fori_loop(0, 10, body, (z, z))
        oa = c_one(dbuf, 30, oa, cb, thresh)
        ob = c_one(dbuf, 61, ob, cb + 512, thresh)
        ob = c_one(dbuf, 62, ob, cb + 512, thresh)
        splice(cb, oa, ob)
        return jnp.max(oa + ob)

    def compact2(d0, d1, thresh):
        def body(i, carry):
            oa0, ob0, oa1, ob1 = carry
            for u in range(3):
                c = i * 3 + u
                oa0 = c_one(d0, c, oa0, 0, thresh)
                oa1 = c_one(d1, c, oa1, CAND_BUF, thresh)
                ob0 = c_one(d0, 31 + c, ob0, 512, thresh)
                ob1 = c_one(d1, 31 + c, ob1, CAND_BUF + 512, thresh)
            return oa0, ob0, oa1, ob1
        z = jnp.zeros((16,), jnp.int32)
        oa0, ob0, oa1, ob1 = lax.fori_loop(0, 10, body, (z, z, z, z))
        oa0 = c_one(d0, 30, oa0, 0, thresh)
        oa1 = c_one(d1, 30, oa1, CAND_BUF, thresh)
        ob0 = c_one(d0, 61, ob0, 512, thresh)
        ob1 = c_one(d1, 61, ob1, CAND_BUF + 512, thresh)
        ob0 = c_one(d0, 62, ob0, 512, thresh)
        ob1 = c_one(d1, 62, ob1, CAND_BUF + 512, thresh)
        splice(0, oa0, ob0)
        splice(CAND_BUF, oa1, ob1)
        return jnp.max(oa0 + ob0), jnp.max(oa1 + ob1)

    def resolve(dbuf, cb, tg, m):
        def exact_fn(_):
            for j in range(NBKT):
                hist[pl.ds(j * 16, 16)] = zi

            def hist_body(c, _):
                v = dbuf[pl.ds(c * 16, 16)]
                bkt = (v * float(NBKT)).astype(jnp.int32)
                addr = bkt * 16 + iota
                lanemask = (c * 16 + iota) < N
                plsc.addupdate_scatter(hist, [addr], ones16, mask=lanemask)
                return 0
            lax.fori_loop(0, 63, hist_body, 0)

            def scan_body(j, carry):
                cum, bstar = carry
                h = hist[pl.ds(j * 16, 16)]
                cum2 = cum + jnp.sum(h)
                hit = jnp.logical_and(bstar < 0, cum2 >= K)
                bstar = jnp.where(hit, j, bstar)
                return cum2, bstar
            _, bstar = lax.fori_loop(0, NBKT, scan_body,
                                     (jnp.int32(0), jnp.int32(-1)))
            bnd = (bstar + 1).astype(jnp.float32) * (1.0 / NBKT)
            prefill(cb)
            return compact1(dbuf, cb, bnd)

        def retry_fn(_):
            mf = jnp.maximum(m.astype(jnp.float32), 1.0)
            ratio = jnp.full((16,), 128.0) / jnp.full((16,), mf)
            tg2 = jnp.full((16,), tg) * jnp.clip(ratio, 0.25, 8.0)
            prefill(cb)
            m2 = compact1(dbuf, cb, tg2)
            ok2 = jnp.logical_and(m2 >= K, m2 <= CAND_CAP)
            return lax.cond(ok2, lambda _: m2, exact_fn, 0)

        ok = jnp.logical_and(m >= K, m <= CAND_CAP)
        return lax.cond(ok, lambda _: m, retry_fn, 0)

    def gather_pairs(dbuf, cb):
        pairs = []
        for j in range(NV):
            iv = cand[pl.ds(cb + j * 16, 16)]
            kv = plsc.load_gather(dbuf, [iv])
            pairs.append((kv, iv))
        return pairs + [None] * (16 - NV)

    def finish_row(b, snet, m_f):
        dbuf, tbuf = dbufs[b], tbufs[b]
        sdbuf, thbuf, ixbuf = sdbufs[b], thbufs[b], ixbufs[b]

        shift_iota = jnp.minimum(iota + 1, 15)
        tie = jnp.zeros((16,), jnp.bool_)
        prev_max = None
        for j in range(7):
            kj = snet[j][0]
            sh = jnp.take(kj, shift_iota)
            eq = jnp.logical_and(kj == sh, kj < jnp.inf)
            tie = jnp.logical_or(tie, jnp.logical_and(eq, iota < 15))
            if prev_max is not None:
                beq = jnp.logical_and(prev_max == jnp.min(kj),
                                      prev_max < jnp.inf)
                tie = jnp.logical_or(tie, jnp.full((16,), beq))
            prev_max = jnp.max(kj)
        n_tie = plsc.all_reduce_population_count(tie)
        bad = jnp.logical_or(jnp.logical_or(m_f > CAND_CAP, m_f < K),
                             jnp.sum(n_tie) > 0)

        for j in range(7):
            sdbuf[pl.ds(j * 16, 16)] = snet[j][0]
            ixbuf[pl.ds(j * 16, 16)] = snet[j][1]

        @pl.when(bad)
        def _fallback():
            def sel_body(k, _):
                def min_body(c, acc):
                    return jnp.minimum(acc, dbuf[pl.ds(c * 16, 16)])
                macc = lax.fori_loop(0, 63, min_body, infv)
                mn = jnp.min(macc)

                def pos_body(c, acc):
                    v = dbuf[pl.ds(c * 16, 16)]
                    cnd = jnp.where(v == mn, c * 16 + iota, NPAD)
                    return jnp.minimum(acc, cnd)
                pacc = lax.fori_loop(0, 63, pos_body,
                                     jnp.full((16,), NPAD, jnp.int32))
                p = jnp.min(pacc)
                lane0 = iota == 0
                kvec = jnp.full((16,), k, jnp.int32)
                plsc.store_scatter(sdbuf, [kvec], jnp.full((16,), mn), mask=lane0)
                plsc.store_scatter(ixbuf, [kvec], jnp.full((16,), p, jnp.int32),
                                   mask=lane0)
                plsc.store_scatter(dbuf, [jnp.full((16,), p, jnp.int32)], infv,
                                   mask=lane0)
                return 0
            lax.fori_loop(0, K, sel_body, 0)

        ix6 = jnp.where(iota < 4, ixbuf[pl.ds(96, 16)], 0)
        ixbuf[pl.ds(96, 16)] = ix6
        sd6 = jnp.where(iota < 4, sdbuf[pl.ds(96, 16)], 0.0)
        mx = jnp.max(sd6)
        rmx = jnp.full((16,), 1.0, jnp.float32) / jnp.full((16,), mx)
        for j in range(7):
            iv = ix6 if j == 6 else ixbuf[pl.ds(j * 16, 16)]
            tv = plsc.load_gather(tbuf, [iv])
            if j == 6:
                tv = jnp.where(iota < 4, tv, 0.0)
            thbuf[pl.ds(j * 16, 16)] = tv
            sv = sd6 if j == 6 else sdbuf[pl.ds(j * 16, 16)]
            sdbuf[pl.ds(j * 16, 16)] = sv * rmx
        return mx

    def process_pair(b0, b1, tg):
        d0, d1 = dbufs[b0], dbufs[b1]
        prefill(0)
        prefill(CAND_BUF)
        m0, m1 = compact2(d0, d1, tg)
        m_f0 = resolve(d0, 0, tg, m0)
        m_f1 = resolve(d1, CAND_BUF, tg, m1)
        ops = _jnp_ops()
        pairs0 = gather_pairs(d0, 0)
        pairs1 = gather_pairs(d1, CAND_BUF)
        snet0 = _sort_network(pairs0, ops, keep=7)
        snet1 = _sort_network(pairs1, ops, keep=7)
        finish_row(b0, snet0, m_f0)
        mx1 = finish_row(b1, snet1, m_f1)
        return mx1 * 1.25

    def loop_body(i, tg):
        r0 = i * NBUF
        for h in range(2):
            b0, b1 = 2 * h, 2 * h + 1
            ra = r0 + 2 * h
            for b, r in ((b0, ra), (b1, ra + 1)):
                for c in in_copies(b, r):
                    c.wait()

                @pl.when(r >= NBUF)
                def _wait_out(b=b, r=r):
                    for c in out_copies(b, r - NBUF):
                        c.wait()

            tg = process_pair(b0, b1, tg)
            for b, r in ((b0, ra), (b1, ra + 1)):
                for c in out_copies(b, r):
                    c.start()

                @pl.when(r + NBUF < ROWS_PER_W)
                def _prefetch(b=b, r=r):
                    for c in in_copies(b, r + NBUF):
                        c.start()
        return tg

    lax.fori_loop(0, ROWS_PER_W // NBUF, loop_body, jnp.float32(2.0))
    for b in range(NBUF):
        for c in out_copies(b, ROWS_PER_W - NBUF + b):
            c.wait()


def _make_topk():
    mesh = plsc.VectorSubcoreMesh(core_axis_name="c", subcore_axis_name="s")
    scratch = ([pltpu.VMEM((NPAD,), jnp.float32)] * NBUF +
               [pltpu.VMEM((N,), jnp.float32)] * NBUF +
               [pltpu.VMEM((OUTW,), jnp.float32)] * NBUF +
               [pltpu.VMEM((OUTW,), jnp.float32)] * NBUF +
               [pltpu.VMEM((OUTW,), jnp.int32)] * NBUF +
               [pltpu.VMEM((2 * CAND_BUF,), jnp.int32),
                pltpu.VMEM((NBKT * 16,), jnp.int32)] +
               [pltpu.SemaphoreType.DMA] * (2 * NBUF))
    return pl.kernel(
        _topk_kernel,
        mesh=mesh,
        out_type=[jax.ShapeDtypeStruct((B, OUTW), jnp.float32),
                  jax.ShapeDtypeStruct((B, OUTW), jnp.float32),
                  jax.ShapeDtypeStruct((B, OUTW), jnp.int32)],
        scratch_types=scratch,
        compiler_params=pltpu.CompilerParams(needs_layout_passes=False, use_tc_tiling_on_sc=False),
    )


def _mlp_kernel(sd_ref, th_ref, ins_ref, w1a_ref, w1b_ref, w1c_ref, b1_ref,
                w2_ref, b2_ref, out_ref):
    sd = sd_ref[...]
    edge = jnp.dot(sd, w1a_ref[...], preferred_element_type=jnp.float32)
    edge += jnp.dot(th_ref[...], w1b_ref[...], preferred_element_type=jnp.float32)
    edge += jnp.dot(ins_ref[...], w1c_ref[...], preferred_element_type=jnp.float32)
    edge += b1_ref[...]
    out = jnp.dot(edge, w2_ref[...], preferred_element_type=jnp.float32)
    out_ref[...] = out + b2_ref[...] - sd


def _mlp(sd, th, ins2, w1a, w1b, w1c, b1, w2p, b2p):
    grid = (B // TC_ROWS,)
    return pl.pallas_call(
        _mlp_kernel,
        grid=grid,
        in_specs=[
            pl.BlockSpec((TC_ROWS, OUTW), lambda i: (i, 0)),
            pl.BlockSpec((TC_ROWS, OUTW), lambda i: (i, 0)),
            pl.BlockSpec((TC_ROWS, 2), lambda i: (i, 0)),
            pl.BlockSpec((OUTW, EMB), lambda i: (0, 0)),
            pl.BlockSpec((OUTW, EMB), lambda i: (0, 0)),
            pl.BlockSpec((2, EMB), lambda i: (0, 0)),
            pl.BlockSpec((1, EMB), lambda i: (0, 0)),
            pl.BlockSpec((EMB, OUTW), lambda i: (0, 0)),
            pl.BlockSpec((1, OUTW), lambda i: (0, 0)),
        ],
        out_specs=pl.BlockSpec((TC_ROWS, OUTW), lambda i: (i, 0)),
        out_shape=jax.ShapeDtypeStruct((B, OUTW), jnp.float32),
    )(sd, th, ins2, w1a, w1b, w1c, b1, w2p, b2p)


def _scatter_kernel(val_hbm, ix_hbm, om_hbm, *scr):
    rbufs = scr[0:NBUF]
    vbufs = scr[NBUF:2 * NBUF]
    ibufs = scr[2 * NBUF:3 * NBUF]
    sbufs = scr[3 * NBUF:4 * NBUF]
    sem_in = scr[4 * NBUF:5 * NBUF]
    sem_out = scr[5 * NBUF:6 * NBUF]

    wid = lax.axis_index("s") * NC + lax.axis_index("c")
    base = wid * ROWS_PER_W
    iota = _iota16()
    pen = jnp.full((16,), PENALTY, jnp.float32)

    def in_copies(b, r):
        return (pltpu.make_async_copy(val_hbm.at[base + r], vbufs[b], sem_in[b]),
                pltpu.make_async_copy(ix_hbm.at[base + r], ibufs[b], sem_in[b]))

    def out_copy(b, r):
        return pltpu.make_async_copy(rbufs[b].at[pl.ds(0, N)],
                                     om_hbm.at[base + r], sem_out[b])

    for b in range(NBUF):
        for j in range(63):
            rbufs[b][pl.ds(j * 16, 16)] = pen
        for c in in_copies(b, b):
            c.start()

    def loop_body(i, _):
        r0 = i * NBUF
        for b in range(NBUF):
            r = r0 + b
            for c in in_copies(b, r):
                c.wait()

            @pl.when(r >= NBUF)
            def _wait_out():
                out_copy(b, r - NBUF).wait()
                for j in range(7):
                    siv = sbufs[b][pl.ds(j * 16, 16)]
                    mask = (j * 16 + iota) < K
                    plsc.store_scatter(rbufs[b], [siv], pen, mask=mask)

            for j in range(7):
                iv = ibufs[b][pl.ds(j * 16, 16)]
                vv = vbufs[b][pl.ds(j * 16, 16)]
                mask = (j * 16 + iota) < K
                plsc.store_scatter(rbufs[b], [iv], vv, mask=mask)
                sbufs[b][pl.ds(j * 16, 16)] = iv
            out_copy(b, r).start()

            @pl.when(r + NBUF < ROWS_PER_W)
            def _prefetch():
                for c in in_copies(b, r + NBUF):
                    c.start()
        return 0

    lax.fori_loop(0, ROWS_PER_W // NBUF, loop_body, 0)
    for b in range(NBUF):
        out_copy(b, ROWS_PER_W - NBUF + b).wait()


def _make_scatter():
    mesh = plsc.VectorSubcoreMesh(core_axis_name="c", subcore_axis_name="s")
    scratch = ([pltpu.VMEM((NPAD,), jnp.float32)] * NBUF +
               [pltpu.VMEM((OUTW,), jnp.float32)] * NBUF +
               [pltpu.VMEM((OUTW,), jnp.int32)] * NBUF +
               [pltpu.VMEM((OUTW,), jnp.int32)] * NBUF +
               [pltpu.SemaphoreType.DMA] * (2 * NBUF))
    return pl.kernel(
        _scatter_kernel,
        mesh=mesh,
        out_type=jax.ShapeDtypeStruct((B, N), jnp.float32),
        scratch_types=scratch,
        compiler_params=pltpu.CompilerParams(needs_layout_passes=False, use_tc_tiling_on_sc=False),
    )


def kernel(theta, dist, ins_feature, W1, b1, W2, b2):
    sd, th, ix = _make_topk()(dist, theta)

    ins2 = jnp.concatenate([ins_feature[0], ins_feature[1]], axis=1)
    w1a = jnp.pad(W1[0:K], ((0, OUTW - K), (0, 0)))
    w1b = jnp.pad(W1[K:2 * K], ((0, OUTW - K), (0, 0)))
    w1c = W1[2 * K:2 * K + 2]
    w2p = jnp.pad(W2, ((0, 0), (0, OUTW - K)))
    b2p = jnp.pad(b2, (0, OUTW - K)).reshape(1, OUTW)
    b1r = b1.reshape(1, EMB)

    out = _mlp(sd, th, ins2, w1a, w1b, w1c, b1r, w2p, b2p)
    return _make_scatter()(out, ix)


# speedup vs baseline: 1.1127x; 1.1127x over previous
import jax
import jax.numpy as jnp
from jax import lax
from jax.experimental import pallas as pl
from jax.experimental.pallas import tpu as pltpu
from jax.experimental.pallas import tpu_sc as plsc

B = 16384
N = 1000
K = 100
EMB = 512
PENALTY = 10.0
NPAD = 1008
OUTW = 128
NBUF = 4
NBKT = 32
CAND_CAP = 160
CAND_BUF = 1024
NC = 2
NS = 16
NW = NC * NS
ROWS_PER_W = B // NW
TC_ROWS = 256


def _iota16():
    return lax.broadcasted_iota(jnp.int32, (16,), 0)


class _Ops:
    def __init__(self, vsort1, rev, minimum, maximum, where, le):
        self.vsort1_f = vsort1
        self.rev_f = rev
        self.minimum = minimum
        self.maximum = maximum
        self.where = where
        self.le = le

    def vsort1(self, pair):
        if pair is None:
            return None
        return self.vsort1_f(pair)

    def rev(self, pair):
        if pair is None:
            return None
        return (self.rev_f(pair[0]), self.rev_f(pair[1]))

    def minmax(self, a, b):
        if a is None and b is None:
            return None, None
        if b is None:
            return a, None
        if a is None:
            return b, None
        ka, pa = a
        kb, pb = b
        m = self.le(ka, kb)
        lo = (self.minimum(ka, kb), self.where(m, pa, pb))
        hi = (self.maximum(ka, kb), self.where(m, pb, pa))
        return lo, hi

    def min_only(self, a, b):
        if b is None:
            return a
        if a is None:
            return b
        ka, pa = a
        kb, pb = b
        m = self.le(ka, kb)
        return (self.minimum(ka, kb), self.where(m, pa, pb))


def _bitonic(Z, ops, keep=None):
    n = len(Z)
    if keep is not None and keep <= 0:
        return [None] * n
    if n == 1:
        return [ops.vsort1(Z[0])]
    half = n // 2
    prune_hi = keep is not None and keep <= half
    lo, hi = [], []
    for i in range(half):
        if prune_hi:
            lo.append(ops.min_only(Z[i], Z[i + half]))
            hi.append(None)
        else:
            l, h = ops.minmax(Z[i], Z[i + half])
            lo.append(l)
            hi.append(h)
    khi = None if keep is None else keep - half
    return _bitonic(lo, ops, keep) + _bitonic(hi, ops, khi)


def _merge(A, Bl, ops, keep=None):
    n = 1
    while n < max(len(A), len(Bl)):
        n *= 2
    A = A + [None] * (n - len(A))
    Bl = Bl + [None] * (n - len(Bl))
    Z = A + [ops.rev(x) for x in reversed(Bl)]
    return _bitonic(Z, ops, keep)


def _sort_network(pairs, ops, keep=None):
    if len(pairs) == 1:
        return [ops.vsort1(pairs[0])]
    h = len(pairs) // 2
    return _merge(_sort_network(pairs[:h], ops),
                  _sort_network(pairs[h:], ops), ops, keep)


def _jnp_ops():
    def vsort1(pair):
        k, p = pair
        ks, ps = plsc.sort_key_val(k, p)
        return (ks, ps)
    return _Ops(vsort1, lambda v: lax.rev(v, (0,)), jnp.minimum,
                jnp.maximum, jnp.where, lambda a, b: a <= b)


def _topk_kernel(dist_hbm, theta_hbm, sd_hbm, th_hbm, ix_hbm, *scr):
    dbufs = scr[0:NBUF]
    tbufs = scr[NBUF:2 * NBUF]
    sdbufs = scr[2 * NBUF:3 * NBUF]
    thbufs = scr[3 * NBUF:4 * NBUF]
    ixbufs = scr[4 * NBUF:5 * NBUF]
    cand = scr[5 * NBUF]
    hist = scr[5 * NBUF + 1]
    sem_in = scr[5 * NBUF + 2:5 * NBUF + 2 + NBUF]
    sem_out = scr[5 * NBUF + 2 + NBUF:5 * NBUF + 2 + 2 * NBUF]

    wid = lax.axis_index("s") * NC + lax.axis_index("c")
    base = wid * ROWS_PER_W
    iota = _iota16()
    ones16 = jnp.full((16,), 1, jnp.int32)
    infv = jnp.full((16,), jnp.inf, jnp.float32)
    zf = jnp.zeros((16,), jnp.float32)
    zi = jnp.zeros((16,), jnp.int32)

    def in_copies(b, r):
        return (pltpu.make_async_copy(dist_hbm.at[base + r],
                                      dbufs[b].at[pl.ds(0, N)], sem_in[b]),
                pltpu.make_async_copy(theta_hbm.at[base + r],
                                      tbufs[b], sem_in[b]))

    def out_copies(b, r):
        return (pltpu.make_async_copy(sdbufs[b], sd_hbm.at[base + r], sem_out[b]),
                pltpu.make_async_copy(thbufs[b], th_hbm.at[base + r], sem_out[b]),
                pltpu.make_async_copy(ixbufs[b], ix_hbm.at[base + r], sem_out[b]))

    for b in range(NBUF):
        dbufs[b][pl.ds(992, 16)] = infv
        sdbufs[b][pl.ds(112, 16)] = zf
        thbufs[b][pl.ds(112, 16)] = zf
        ixbufs[b][pl.ds(112, 16)] = zi
        for c in in_copies(b, b):
            c.start()

    lane15 = jnp.full((16,), 15, jnp.int32)
    NV = CAND_CAP // 16

    def prefill(cb):
        for j in range(NV):
            cand[pl.ds(cb + j * 16, 16)] = jnp.full((16,), NPAD - 1, jnp.int32)

    def c_one(dbuf, c, off_v, cbase, thresh):
        v = dbuf[pl.ds(c * 16, 16)]
        sel = v < thresh
        cnt = jnp.where(sel, 1, 0)
        pfx = jnp.cumsum(cnt)
        pos = pfx - 1 + off_v
        if cbase:
            pos = pos + cbase
        plsc.store_scatter(cand, [pos], c * 16 + iota, mask=sel)
        return off_v + jnp.take(pfx, lane15)

    def splice(cb, oa, ob):
        basev = oa + iota + cb
        for j in range(NV):
            iv = cand[pl.ds(cb + 512 + j * 16, 16)]
            ivm = jnp.where(j * 16 + iota < ob, iv, NPAD - 1)
            plsc.store_scatter(cand, [basev + j * 16], ivm)

    def compact1(dbuf, cb, thresh):
        def body(i, carry):
            oa, ob = carry
            for u in range(3):
                oa = c_one(dbuf, i * 3 + u, oa, cb, thresh)
                ob = c_one(dbuf, 31 + i * 3 + u, ob, cb + 512, thresh)
            return oa, ob
        z = jnp.zeros((16,), jnp.int32)
        oa, ob = lax.fori_loop(0, 10, body, (z, z))
        oa = c_one(dbuf, 30, oa, cb, thresh)
        ob = c_one(dbuf, 61, ob, cb + 512, thresh)
        ob = c_one(dbuf, 62, ob, cb + 512, thresh)
        splice(cb, oa, ob)
        return jnp.max(oa + ob)

    def compact2(d0, d1, thresh):
        def body(i, carry):
            oa0, ob0, oa1, ob1 = carry
            for u in range(3):
                c = i * 3 + u
                oa0 = c_one(d0, c, oa0, 0, thresh)
                oa1 = c_one(d1, c, oa1, CAND_BUF, thresh)
                ob0 = c_one(d0, 31 + c, ob0, 512, thresh)
                ob1 = c_one(d1, 31 + c, ob1, CAND_BUF + 512, thresh)
            return oa0, ob0, oa1, ob1
        z = jnp.zeros((16,), jnp.int32)
        oa0, ob0, oa1, ob1 = lax.fori_loop(0, 10, body, (z, z, z, z))
        oa0 = c_one(d0, 30, oa0, 0, thresh)
        oa1 = c_one(d1, 30, oa1, CAND_BUF, thresh)
        ob0 = c_one(d0, 61, ob0, 512, thresh)
        ob1 = c_one(d1, 61, ob1, CAND_BUF + 512, thresh)
        ob0 = c_one(d0, 62, ob0, 512, thresh)
        ob1 = c_one(d1, 62, ob1, CAND_BUF + 512, thresh)
        splice(0, oa0, ob0)
        splice(CAND_BUF, oa1, ob1)
        return jnp.max(oa0 + ob0), jnp.max(oa1 + ob1)

    def resolve(dbuf, cb, tg, m):
        def exact_fn(_):
            for j in range(NBKT):
                hist[pl.ds(j * 16, 16)] = zi

            def hist_body(c, _):
                v = dbuf[pl.ds(c * 16, 16)]
                bkt = (v * float(NBKT)).astype(jnp.int32)
                addr = bkt * 16 + iota
                lanemask = (c * 16 + iota) < N
                plsc.addupdate_scatter(hist, [addr], ones16, mask=lanemask)
                return 0
            lax.fori_loop(0, 63, hist_body, 0)

            def scan_body(j, carry):
                cum, bstar = carry
                h = hist[pl.ds(j * 16, 16)]
                cum2 = cum + jnp.sum(h)
                hit = jnp.logical_and(bstar < 0, cum2 >= K)
                bstar = jnp.where(hit, j, bstar)
                return cum2, bstar
            _, bstar = lax.fori_loop(0, NBKT, scan_body,
                                     (jnp.int32(0), jnp.int32(-1)))
            bnd = (bstar + 1).astype(jnp.float32) * (1.0 / NBKT)
            prefill(cb)
            return compact1(dbuf, cb, bnd)

        def retry_fn(_):
            mf = jnp.maximum(m.astype(jnp.float32), 1.0)
            ratio = jnp.full((16,), 128.0) / jnp.full((16,), mf)
            tg2 = jnp.full((16,), tg) * jnp.clip(ratio, 0.25, 8.0)
            prefill(cb)
            m2 = compact1(dbuf, cb, tg2)
            ok2 = jnp.logical_and(m2 >= K, m2 <= CAND_CAP)
            return lax.cond(ok2, lambda _: m2, exact_fn, 0)

        ok = jnp.logical_and(m >= K, m <= CAND_CAP)
        return lax.cond(ok, lambda _: m, retry_fn, 0)

    def gather_pairs(dbuf, cb):
        pairs = []
        for j in range(NV):
            iv = cand[pl.ds(cb + j * 16, 16)]
            kv = plsc.load_gather(dbuf, [iv])
            pairs.append((kv, iv))
        return pairs + [None] * (16 - NV)

    def finish_row(b, snet, m_f):
        dbuf, tbuf = dbufs[b], tbufs[b]
        sdbuf, thbuf, ixbuf = sdbufs[b], thbufs[b], ixbufs[b]

        shift_iota = jnp.minimum(iota + 1, 15)
        tie = jnp.zeros((16,), jnp.bool_)
        prev_max = None
        for j in range(7):
            kj = snet[j][0]
            sh = jnp.take(kj, shift_iota)
            eq = jnp.logical_and(kj == sh, kj < jnp.inf)
            tie = jnp.logical_or(tie, jnp.logical_and(eq, iota < 15))
            if prev_max is not None:
                beq = jnp.logical_and(prev_max == jnp.min(kj),
                                      prev_max < jnp.inf)
                tie = jnp.logical_or(tie, jnp.full((16,), beq))
            prev_max = jnp.max(kj)
        n_tie = plsc.all_reduce_population_count(tie)
        bad = jnp.logical_or(jnp.logical_or(m_f > CAND_CAP, m_f < K),
                             jnp.sum(n_tie) > 0)

        for j in range(7):
            sdbuf[pl.ds(j * 16, 16)] = snet[j][0]
            ixbuf[pl.ds(j * 16, 16)] = snet[j][1]

        @pl.when(bad)
        def _fallback():
            def sel_body(k, _):
                def min_body(c, acc):
                    return jnp.minimum(acc, dbuf[pl.ds(c * 16, 16)])
                macc = lax.fori_loop(0, 63, min_body, infv)
                mn = jnp.min(macc)

                def pos_body(c, acc):
                    v = dbuf[pl.ds(c * 16, 16)]
                    cnd = jnp.where(v == mn, c * 16 + iota, NPAD)
                    return jnp.minimum(acc, cnd)
                pacc = lax.fori_loop(0, 63, pos_body,
                                     jnp.full((16,), NPAD, jnp.int32))
                p = jnp.min(pacc)
                lane0 = iota == 0
                kvec = jnp.full((16,), k, jnp.int32)
                plsc.store_scatter(sdbuf, [kvec], jnp.full((16,), mn), mask=lane0)
                plsc.store_scatter(ixbuf, [kvec], jnp.full((16,), p, jnp.int32),
                                   mask=lane0)
                plsc.store_scatter(dbuf, [jnp.full((16,), p, jnp.int32)], infv,
                                   mask=lane0)
                return 0
            lax.fori_loop(0, K, sel_body, 0)

        ix6 = jnp.where(iota < 4, ixbuf[pl.ds(96, 16)], 0)
        ixbuf[pl.ds(96, 16)] = ix6
        sd6 = jnp.where(iota < 4, sdbuf[pl.ds(96, 16)], 0.0)
        mx = jnp.max(sd6)
        rmx = jnp.full((16,), 1.0, jnp.float32) / jnp.full((16,), mx)
        for j in range(7):
            iv = ix6 if j == 6 else ixbuf[pl.ds(j * 16, 16)]
            tv = plsc.load_gather(tbuf, [iv])
            if j == 6:
                tv = jnp.where(iota < 4, tv, 0.0)
            thbuf[pl.ds(j * 16, 16)] = tv
            sv = sd6 if j == 6 else sdbuf[pl.ds(j * 16, 16)]
            sdbuf[pl.ds(j * 16, 16)] = sv * rmx
        return mx

    def process_pair(b0, b1, tg):
        d0, d1 = dbufs[b0], dbufs[b1]
        prefill(0)
        prefill(CAND_BUF)
        m0, m1 = compact2(d0, d1, tg)
        m_f0 = resolve(d0, 0, tg, m0)
        m_f1 = resolve(d1, CAND_BUF, tg, m1)
        ops = _jnp_ops()
        pairs0 = gather_pairs(d0, 0)
        pairs1 = gather_pairs(d1, CAND_BUF)
        snet0 = _sort_network(pairs0, ops, keep=7)
        snet1 = _sort_network(pairs1, ops, keep=7)
        finish_row(b0, snet0, m_f0)
        mx1 = finish_row(b1, snet1, m_f1)
        return mx1 * 1.25

    def loop_body(i, tg):
        r0 = i * NBUF
        for h in range(2):
            b0, b1 = 2 * h, 2 * h + 1
            ra = r0 + 2 * h
            for b, r in ((b0, ra), (b1, ra + 1)):
                for c in in_copies(b, r):
                    c.wait()

                @pl.when(r >= NBUF)
                def _wait_out(b=b, r=r):
                    for c in out_copies(b, r - NBUF):
                        c.wait()

            tg = process_pair(b0, b1, tg)
            for b, r in ((b0, ra), (b1, ra + 1)):
                for c in out_copies(b, r):
                    c.start()

                @pl.when(r + NBUF < ROWS_PER_W)
                def _prefetch(b=b, r=r):
                    for c in in_copies(b, r + NBUF):
                        c.start()
        return tg

    lax.fori_loop(0, ROWS_PER_W // NBUF, loop_body, jnp.float32(2.0))
    for b in range(NBUF):
        for c in out_copies(b, ROWS_PER_W - NBUF + b):
            c.wait()


def _make_topk():
    mesh = plsc.VectorSubcoreMesh(core_axis_name="c", subcore_axis_name="s")
    scratch = ([pltpu.VMEM((NPAD,), jnp.float32)] * NBUF +
               [pltpu.VMEM((N,), jnp.float32)] * NBUF +
               [pltpu.VMEM((OUTW,), jnp.float32)] * NBUF +
               [pltpu.VMEM((OUTW,), jnp.float32)] * NBUF +
               [pltpu.VMEM((OUTW,), jnp.int32)] * NBUF +
               [pltpu.VMEM((2 * CAND_BUF,), jnp.int32),
                pltpu.VMEM((NBKT * 16,), jnp.int32)] +
               [pltpu.SemaphoreType.DMA] * (2 * NBUF))
    return pl.kernel(
        _topk_kernel,
        mesh=mesh,
        out_type=[jax.ShapeDtypeStruct((B, OUTW), jnp.float32),
                  jax.ShapeDtypeStruct((B, OUTW), jnp.float32),
                  jax.ShapeDtypeStruct((B, OUTW), jnp.int32)],
        scratch_types=scratch,
        compiler_params=pltpu.CompilerParams(needs_layout_passes=False, use_tc_tiling_on_sc=False),
    )


def _mlp_kernel(sd_ref, th_ref, ins_ref, w1a_ref, w1b_ref, w1c_ref, b1_ref,
                w2_ref, b2_ref, out_ref):
    sd = sd_ref[...]
    edge = jnp.dot(sd, w1a_ref[...], preferred_element_type=jnp.float32)
    edge += jnp.dot(th_ref[...], w1b_ref[...], preferred_element_type=jnp.float32)
    edge += jnp.dot(ins_ref[...], w1c_ref[...], preferred_element_type=jnp.float32)
    edge += b1_ref[...]
    out = jnp.dot(edge, w2_ref[...], preferred_element_type=jnp.float32)
    out_ref[...] = out + b2_ref[...] - sd


def _mlp(sd, th, ins2, w1a, w1b, w1c, b1, w2p, b2p):
    grid = (B // TC_ROWS,)
    return pl.pallas_call(
        _mlp_kernel,
        grid=grid,
        in_specs=[
            pl.BlockSpec((TC_ROWS, OUTW), lambda i: (i, 0)),
            pl.BlockSpec((TC_ROWS, OUTW), lambda i: (i, 0)),
            pl.BlockSpec((TC_ROWS, 2), lambda i: (i, 0)),
            pl.BlockSpec((OUTW, EMB), lambda i: (0, 0)),
            pl.BlockSpec((OUTW, EMB), lambda i: (0, 0)),
            pl.BlockSpec((2, EMB), lambda i: (0, 0)),
            pl.BlockSpec((1, EMB), lambda i: (0, 0)),
            pl.BlockSpec((EMB, OUTW), lambda i: (0, 0)),
            pl.BlockSpec((1, OUTW), lambda i: (0, 0)),
        ],
        out_specs=pl.BlockSpec((TC_ROWS, OUTW), lambda i: (i, 0)),
        out_shape=jax.ShapeDtypeStruct((B, OUTW), jnp.float32),
    )(sd, th, ins2, w1a, w1b, w1c, b1, w2p, b2p)


def _scatter_kernel(val_hbm, ix_hbm, om_hbm, *scr):
    rbufs = scr[0:NBUF]
    vbufs = scr[NBUF:2 * NBUF]
    ibufs = scr[2 * NBUF:3 * NBUF]
    sbufs = scr[3 * NBUF:4 * NBUF]
    sem_in = scr[4 * NBUF:5 * NBUF]
    sem_out = scr[5 * NBUF:6 * NBUF]

    wid = lax.axis_index("s") * NC + lax.axis_index("c")
    base = wid * ROWS_PER_W
    iota = _iota16()
    pen = jnp.full((16,), PENALTY, jnp.float32)

    def in_copies(b, r):
        return (pltpu.make_async_copy(val_hbm.at[base + r], vbufs[b], sem_in[b]),
                pltpu.make_async_copy(ix_hbm.at[base + r], ibufs[b], sem_in[b]))

    def out_copy(b, r):
        return pltpu.make_async_copy(rbufs[b].at[pl.ds(0, N)],
                                     om_hbm.at[base + r], sem_out[b])

    for b in range(NBUF):
        for j in range(63):
            rbufs[b][pl.ds(j * 16, 16)] = pen
        for c in in_copies(b, b):
            c.start()

    def loop_body(i, _):
        r0 = i * NBUF
        for b in range(NBUF):
            r = r0 + b
            for c in in_copies(b, r):
                c.wait()

            @pl.when(r >= NBUF)
            def _wait_out():
                out_copy(b, r - NBUF).wait()
                for j in range(7):
                    siv = sbufs[b][pl.ds(j * 16, 16)]
                    mask = (j * 16 + iota) < K
                    plsc.store_scatter(rbufs[b], [siv], pen, mask=mask)

            for j in range(7):
                iv = ibufs[b][pl.ds(j * 16, 16)]
                vv = vbufs[b][pl.ds(j * 16, 16)]
                mask = (j * 16 + iota) < K
                plsc.store_scatter(rbufs[b], [iv], vv, mask=mask)
                sbufs[b][pl.ds(j * 16, 16)] = iv
            out_copy(b, r).start()

            @pl.when(r + NBUF < ROWS_PER_W)
            def _prefetch():
                for c in in_copies(b, r + NBUF):
                    c.start()
        return 0

    lax.fori_loop(0, ROWS_PER_W // NBUF, loop_body, 0)
    for b in range(NBUF):
        out_copy(b, ROWS_PER_W - NBUF + b).wait()


def _make_scatter():
    mesh = plsc.VectorSubcoreMesh(core_axis_name="c", subcore_axis_name="s")
    scratch = ([pltpu.VMEM((NPAD,), jnp.float32)] * NBUF +
               [pltpu.VMEM((OUTW,), jnp.float32)] * NBUF +
               [pltpu.VMEM((OUTW,), jnp.int32)] * NBUF +
               [pltpu.VMEM((OUTW,), jnp.int32)] * NBUF +
               [pltpu.SemaphoreType.DMA] * (2 * NBUF))
    return pl.kernel(
        _scatter_kernel,
        mesh=mesh,
        out_type=jax.ShapeDtypeStruct((B, N), jnp.float32),
        scratch_types=scratch,
        compiler_params=pltpu.CompilerParams(needs_layout_passes=False, use_tc_tiling_on_sc=False),
    )


def kernel(theta, dist, ins_feature, W1, b1, W2, b2):
    sd, th, ix = _make_topk()(dist, theta)

    ins2 = jnp.concatenate([ins_feature[0], ins_feature[1]], axis=1)
    w1a = jnp.pad(W1[0:K], ((0, OUTW - K), (0, 0)))
    w1b = jnp.pad(W1[K:2 * K], ((0, OUTW - K), (0, 0)))
    w1c = W1[2 * K:2 * K + 2]
    w2p = jnp.pad(W2, ((0, 0), (0, OUTW - K)))
    b2p = jnp.pad(b2, (0, OUTW - K)).reshape(1, OUTW)
    b1r = b1.reshape(1, EMB)

    out = _mlp(sd, th, ins2, w1a, w1b, w1c, b1r, w2p, b2p)
    return _make_scatter()(out, ix)


# speedup vs baseline: 1.1225x; 1.0089x over previous
import jax
import jax.numpy as jnp
from jax import lax
from jax.experimental import pallas as pl
from jax.experimental.pallas import tpu as pltpu
from jax.experimental.pallas import tpu_sc as plsc

B = 16384
N = 1000
K = 100
EMB = 512
PENALTY = 10.0
NPAD = 1008
OUTW = 128
NBUF = 4
NBKT = 32
CAND_CAP = 192
CAND_BUF = 1024
NC = 2
NS = 16
NW = NC * NS
ROWS_PER_W = B // NW
TC_ROWS = 256


def _iota16():
    return lax.broadcasted_iota(jnp.int32, (16,), 0)


class _Ops:
    def __init__(self, vsort1, rev, minimum, maximum, where, le):
        self.vsort1_f = vsort1
        self.rev_f = rev
        self.minimum = minimum
        self.maximum = maximum
        self.where = where
        self.le = le

    def vsort1(self, pair):
        if pair is None:
            return None
        return self.vsort1_f(pair)

    def rev(self, pair):
        if pair is None:
            return None
        return (self.rev_f(pair[0]), self.rev_f(pair[1]))

    def minmax(self, a, b):
        if a is None and b is None:
            return None, None
        if b is None:
            return a, None
        if a is None:
            return b, None
        ka, pa = a
        kb, pb = b
        m = self.le(ka, kb)
        lo = (self.minimum(ka, kb), self.where(m, pa, pb))
        hi = (self.maximum(ka, kb), self.where(m, pb, pa))
        return lo, hi

    def min_only(self, a, b):
        if b is None:
            return a
        if a is None:
            return b
        ka, pa = a
        kb, pb = b
        m = self.le(ka, kb)
        return (self.minimum(ka, kb), self.where(m, pa, pb))


def _bitonic(Z, ops, keep=None):
    n = len(Z)
    if keep is not None and keep <= 0:
        return [None] * n
    if n == 1:
        return [ops.vsort1(Z[0])]
    half = n // 2
    prune_hi = keep is not None and keep <= half
    lo, hi = [], []
    for i in range(half):
        if prune_hi:
            lo.append(ops.min_only(Z[i], Z[i + half]))
            hi.append(None)
        else:
            l, h = ops.minmax(Z[i], Z[i + half])
            lo.append(l)
            hi.append(h)
    khi = None if keep is None else keep - half
    return _bitonic(lo, ops, keep) + _bitonic(hi, ops, khi)


def _merge(A, Bl, ops, keep=None):
    n = 1
    while n < max(len(A), len(Bl)):
        n *= 2
    A = A + [None] * (n - len(A))
    Bl = Bl + [None] * (n - len(Bl))
    Z = A + [ops.rev(x) for x in reversed(Bl)]
    return _bitonic(Z, ops, keep)


def _sort_network(pairs, ops, keep=None):
    if len(pairs) == 1:
        return [ops.vsort1(pairs[0])]
    h = len(pairs) // 2
    return _merge(_sort_network(pairs[:h], ops),
                  _sort_network(pairs[h:], ops), ops, keep)


def _jnp_ops():
    def vsort1(pair):
        k, p = pair
        ks, ps = plsc.sort_key_val(k, p)
        return (ks, ps)
    return _Ops(vsort1, lambda v: lax.rev(v, (0,)), jnp.minimum,
                jnp.maximum, jnp.where, lambda a, b: a <= b)


def _topk_kernel(dist_hbm, theta_hbm, sd_hbm, th_hbm, ix_hbm, *scr):
    dbufs = scr[0:NBUF]
    tbufs = scr[NBUF:2 * NBUF]
    sdbufs = scr[2 * NBUF:3 * NBUF]
    thbufs = scr[3 * NBUF:4 * NBUF]
    ixbufs = scr[4 * NBUF:5 * NBUF]
    cand = scr[5 * NBUF]
    hist = scr[5 * NBUF + 1]
    sem_in = scr[5 * NBUF + 2:5 * NBUF + 2 + NBUF]
    sem_out = scr[5 * NBUF + 2 + NBUF:5 * NBUF + 2 + 2 * NBUF]

    wid = lax.axis_index("s") * NC + lax.axis_index("c")
    base = wid * ROWS_PER_W
    iota = _iota16()
    ones16 = jnp.full((16,), 1, jnp.int32)
    infv = jnp.full((16,), jnp.inf, jnp.float32)
    zf = jnp.zeros((16,), jnp.float32)
    zi = jnp.zeros((16,), jnp.int32)

    def in_copies(b, r):
        return (pltpu.make_async_copy(dist_hbm.at[base + r],
                                      dbufs[b].at[pl.ds(0, N)], sem_in[b]),
                pltpu.make_async_copy(theta_hbm.at[base + r],
                                      tbufs[b], sem_in[b]))

    def out_copies(b, r):
        return (pltpu.make_async_copy(sdbufs[b], sd_hbm.at[base + r], sem_out[b]),
                pltpu.make_async_copy(thbufs[b], th_hbm.at[base + r], sem_out[b]),
                pltpu.make_async_copy(ixbufs[b], ix_hbm.at[base + r], sem_out[b]))

    for b in range(NBUF):
        dbufs[b][pl.ds(992, 16)] = infv
        sdbufs[b][pl.ds(112, 16)] = zf
        thbufs[b][pl.ds(112, 16)] = zf
        ixbufs[b][pl.ds(112, 16)] = zi
        for c in in_copies(b, b):
            c.start()

    def process_row(b, tg):
        dbuf, tbuf = dbufs[b], tbufs[b]
        sdbuf, thbuf, ixbuf = sdbufs[b], thbufs[b], ixbufs[b]

        def prefill():
            for j in range(CAND_CAP // 16):
                cand[pl.ds(j * 16, 16)] = jnp.full((16,), NPAD - 1, jnp.int32)

        def compact(thresh):
            def body(i, off_v):
                for u in range(3):
                    c = i * 3 + u
                    v = dbuf[pl.ds(c * 16, 16)]
                    sel = v < thresh
                    cnt = jnp.where(sel, 1, 0)
                    pos = jnp.cumsum(cnt) - 1 + off_v
                    plsc.store_scatter(cand, [pos], c * 16 + iota, mask=sel)
                    off_v = off_v + plsc.all_reduce_population_count(sel)
                return off_v
            return lax.fori_loop(0, 21, body, jnp.zeros((16,), jnp.int32))

        prefill()
        m = jnp.max(compact(tg))
        ok = jnp.logical_and(m >= K, m <= CAND_CAP)

        def fast_fn(_):
            return m

        def exact_fn(_):
            for j in range(NBKT):
                hist[pl.ds(j * 16, 16)] = zi

            def hist_body(c, _):
                v = dbuf[pl.ds(c * 16, 16)]
                bkt = (v * float(NBKT)).astype(jnp.int32)
                addr = bkt * 16 + iota
                lanemask = (c * 16 + iota) < N
                plsc.addupdate_scatter(hist, [addr], ones16, mask=lanemask)
                return 0
            lax.fori_loop(0, 63, hist_body, 0)

            def scan_body(j, carry):
                cum, bstar = carry
                h = hist[pl.ds(j * 16, 16)]
                cum2 = cum + jnp.sum(h)
                hit = jnp.logical_and(bstar < 0, cum2 >= K)
                bstar = jnp.where(hit, j, bstar)
                return cum2, bstar
            _, bstar = lax.fori_loop(0, NBKT, scan_body,
                                     (jnp.int32(0), jnp.int32(-1)))
            bnd = (bstar + 1).astype(jnp.float32) * (1.0 / NBKT)
            prefill()
            return jnp.max(compact(bnd))

        m_f = lax.cond(ok, fast_fn, exact_fn, 0)

        ops = _jnp_ops()
        pairs = []
        for j in range(CAND_CAP // 16):
            iv = cand[pl.ds(j * 16, 16)]
            kv = plsc.load_gather(dbuf, [iv])
            pairs.append((kv, iv))
        pairs += [None] * (16 - len(pairs))
        snet = _sort_network(pairs, ops, keep=7)

        shift_iota = jnp.minimum(iota + 1, 15)
        tie = jnp.zeros((16,), jnp.bool_)
        prev_max = None
        for j in range(7):
            kj = snet[j][0]
            sh = jnp.take(kj, shift_iota)
            eq = jnp.logical_and(kj == sh, kj < jnp.inf)
            tie = jnp.logical_or(tie, jnp.logical_and(eq, iota < 15))
            if prev_max is not None:
                beq = jnp.logical_and(prev_max == jnp.min(kj),
                                      prev_max < jnp.inf)
                tie = jnp.logical_or(tie, jnp.full((16,), beq))
            prev_max = jnp.max(kj)
        n_tie = plsc.all_reduce_population_count(tie)
        bad = jnp.logical_or(jnp.logical_or(m_f > CAND_CAP, m_f < K),
                             jnp.sum(n_tie) > 0)

        for j in range(7):
            sdbuf[pl.ds(j * 16, 16)] = snet[j][0]
            ixbuf[pl.ds(j * 16, 16)] = snet[j][1]

        @pl.when(bad)
        def _fallback():
            def sel_body(k, _):
                def min_body(c, acc):
                    return jnp.minimum(acc, dbuf[pl.ds(c * 16, 16)])
                macc = lax.fori_loop(0, 63, min_body, infv)
                mn = jnp.min(macc)

                def pos_body(c, acc):
                    v = dbuf[pl.ds(c * 16, 16)]
                    cnd = jnp.where(v == mn, c * 16 + iota, NPAD)
                    return jnp.minimum(acc, cnd)
                pacc = lax.fori_loop(0, 63, pos_body,
                                     jnp.full((16,), NPAD, jnp.int32))
                p = jnp.min(pacc)
                lane0 = iota == 0
                kvec = jnp.full((16,), k, jnp.int32)
                plsc.store_scatter(sdbuf, [kvec], jnp.full((16,), mn), mask=lane0)
                plsc.store_scatter(ixbuf, [kvec], jnp.full((16,), p, jnp.int32),
                                   mask=lane0)
                plsc.store_scatter(dbuf, [jnp.full((16,), p, jnp.int32)], infv,
                                   mask=lane0)
                return 0
            lax.fori_loop(0, K, sel_body, 0)

        ix6 = jnp.where(iota < 4, ixbuf[pl.ds(96, 16)], 0)
        ixbuf[pl.ds(96, 16)] = ix6
        sd6 = jnp.where(iota < 4, sdbuf[pl.ds(96, 16)], 0.0)
        mx = jnp.max(sd6)
        for j in range(7):
            iv = ix6 if j == 6 else ixbuf[pl.ds(j * 16, 16)]
            tv = plsc.load_gather(tbuf, [iv])
            if j == 6:
                tv = jnp.where(iota < 4, tv, 0.0)
            thbuf[pl.ds(j * 16, 16)] = tv
            sv = sd6 if j == 6 else sdbuf[pl.ds(j * 16, 16)]
            sdbuf[pl.ds(j * 16, 16)] = sv / mx
        return mx * 1.25

    def loop_body(i, tg):
        r0 = i * NBUF
        for b in range(NBUF):
            r = r0 + b
            for c in in_copies(b, r):
                c.wait()

            @pl.when(r >= NBUF)
            def _wait_out():
                for c in out_copies(b, r - NBUF):
                    c.wait()

            tg = process_row(b, tg)
            for c in out_copies(b, r):
                c.start()

            @pl.when(r + NBUF < ROWS_PER_W)
            def _prefetch():
                for c in in_copies(b, r + NBUF):
                    c.start()
        return tg

    lax.fori_loop(0, ROWS_PER_W // NBUF, loop_body, jnp.float32(2.0))
    for b in range(NBUF):
        for c in out_copies(b, ROWS_PER_W - NBUF + b):
            c.wait()


def _make_topk():
    mesh = plsc.VectorSubcoreMesh(core_axis_name="c", subcore_axis_name="s")
    scratch = ([pltpu.VMEM((NPAD,), jnp.float32)] * NBUF +
               [pltpu.VMEM((N,), jnp.float32)] * NBUF +
               [pltpu.VMEM((OUTW,), jnp.float32)] * NBUF +
               [pltpu.VMEM((OUTW,), jnp.float32)] * NBUF +
               [pltpu.VMEM((OUTW,), jnp.int32)] * NBUF +
               [pltpu.VMEM((CAND_BUF,), jnp.int32),
                pltpu.VMEM((NBKT * 16,), jnp.int32)] +
               [pltpu.SemaphoreType.DMA] * (2 * NBUF))
    return pl.kernel(
        _topk_kernel,
        mesh=mesh,
        out_type=[jax.ShapeDtypeStruct((B, OUTW), jnp.float32),
                  jax.ShapeDtypeStruct((B, OUTW), jnp.float32),
                  jax.ShapeDtypeStruct((B, OUTW), jnp.int32)],
        scratch_types=scratch,
        compiler_params=pltpu.CompilerParams(needs_layout_passes=False, use_tc_tiling_on_sc=False),
    )


def _mlp_kernel(sd_ref, th_ref, ins_ref, w1a_ref, w1b_ref, w1c_ref, b1_ref,
                w2_ref, b2_ref, out_ref):
    sd = sd_ref[...]
    edge = jnp.dot(sd, w1a_ref[...], preferred_element_type=jnp.float32)
    edge += jnp.dot(th_ref[...], w1b_ref[...], preferred_element_type=jnp.float32)
    edge += jnp.dot(ins_ref[...], w1c_ref[...], preferred_element_type=jnp.float32)
    edge += b1_ref[...]
    out = jnp.dot(edge, w2_ref[...], preferred_element_type=jnp.float32)
    out_ref[...] = out + b2_ref[...] - sd


def _mlp(sd, th, ins2, w1a, w1b, w1c, b1, w2p, b2p):
    grid = (B // TC_ROWS,)
    return pl.pallas_call(
        _mlp_kernel,
        grid=grid,
        in_specs=[
            pl.BlockSpec((TC_ROWS, OUTW), lambda i: (i, 0)),
            pl.BlockSpec((TC_ROWS, OUTW), lambda i: (i, 0)),
            pl.BlockSpec((TC_ROWS, 2), lambda i: (i, 0)),
            pl.BlockSpec((OUTW, EMB), lambda i: (0, 0)),
            pl.BlockSpec((OUTW, EMB), lambda i: (0, 0)),
            pl.BlockSpec((2, EMB), lambda i: (0, 0)),
            pl.BlockSpec((1, EMB), lambda i: (0, 0)),
            pl.BlockSpec((EMB, OUTW), lambda i: (0, 0)),
            pl.BlockSpec((1, OUTW), lambda i: (0, 0)),
        ],
        out_specs=pl.BlockSpec((TC_ROWS, OUTW), lambda i: (i, 0)),
        out_shape=jax.ShapeDtypeStruct((B, OUTW), jnp.float32),
    )(sd, th, ins2, w1a, w1b, w1c, b1, w2p, b2p)


def _scatter_kernel(val_hbm, ix_hbm, om_hbm, *scr):
    rbufs = scr[0:NBUF]
    vbufs = scr[NBUF:2 * NBUF]
    ibufs = scr[2 * NBUF:3 * NBUF]
    sem_in = scr[3 * NBUF:4 * NBUF]
    sem_out = scr[4 * NBUF:5 * NBUF]

    wid = lax.axis_index("s") * NC + lax.axis_index("c")
    base = wid * ROWS_PER_W
    iota = _iota16()
    pen = jnp.full((16,), PENALTY, jnp.float32)

    def in_copies(b, r):
        return (pltpu.make_async_copy(val_hbm.at[base + r], vbufs[b], sem_in[b]),
                pltpu.make_async_copy(ix_hbm.at[base + r], ibufs[b], sem_in[b]))

    def out_copy(b, r):
        return pltpu.make_async_copy(rbufs[b].at[pl.ds(0, N)],
                                     om_hbm.at[base + r], sem_out[b])

    for b in range(NBUF):
        for c in in_copies(b, b):
            c.start()

    def loop_body(i, _):
        r0 = i * NBUF
        for b in range(NBUF):
            r = r0 + b
            for c in in_copies(b, r):
                c.wait()

            @pl.when(r >= NBUF)
            def _wait_out():
                out_copy(b, r - NBUF).wait()

            for j in range(63):
                rbufs[b][pl.ds(j * 16, 16)] = pen
            for j in range(7):
                iv = ibufs[b][pl.ds(j * 16, 16)]
                vv = vbufs[b][pl.ds(j * 16, 16)]
                mask = (j * 16 + iota) < K
                plsc.store_scatter(rbufs[b], [iv], vv, mask=mask)
            out_copy(b, r).start()

            @pl.when(r + NBUF < ROWS_PER_W)
            def _prefetch():
                for c in in_copies(b, r + NBUF):
                    c.start()
        return 0

    lax.fori_loop(0, ROWS_PER_W // NBUF, loop_body, 0)
    for b in range(NBUF):
        out_copy(b, ROWS_PER_W - NBUF + b).wait()


def _make_scatter():
    mesh = plsc.VectorSubcoreMesh(core_axis_name="c", subcore_axis_name="s")
    scratch = ([pltpu.VMEM((NPAD,), jnp.float32)] * NBUF +
               [pltpu.VMEM((OUTW,), jnp.float32)] * NBUF +
               [pltpu.VMEM((OUTW,), jnp.int32)] * NBUF +
               [pltpu.SemaphoreType.DMA] * (2 * NBUF))
    return pl.kernel(
        _scatter_kernel,
        mesh=mesh,
        out_type=jax.ShapeDtypeStruct((B, N), jnp.float32),
        scratch_types=scratch,
        compiler_params=pltpu.CompilerParams(needs_layout_passes=False, use_tc_tiling_on_sc=False),
    )


def kernel(theta, dist, ins_feature, W1, b1, W2, b2):
    sd, th, ix = _make_topk()(dist, theta)

    ins2 = jnp.concatenate([ins_feature[0], ins_feature[1]], axis=1)
    w1a = jnp.pad(W1[0:K], ((0, OUTW - K), (0, 0)))
    w1b = jnp.pad(W1[K:2 * K], ((0, OUTW - K), (0, 0)))
    w1c = W1[2 * K:2 * K + 2]
    w2p = jnp.pad(W2, ((0, 0), (0, OUTW - K)))
    b2p = jnp.pad(b2, (0, OUTW - K)).reshape(1, OUTW)
    b1r = b1.reshape(1, EMB)

    out = _mlp(sd, th, ins2, w1a, w1b, w1c, b1r, w2p, b2p)
    return _make_scatter()(out, ix)


# speedup vs baseline: 1.1264x; 1.0035x over previous
import jax
import jax.numpy as jnp
from jax import lax
from jax.experimental import pallas as pl
from jax.experimental.pallas import tpu as pltpu
from jax.experimental.pallas import tpu_sc as plsc

B = 16384
N = 1000
K = 100
EMB = 512
PENALTY = 10.0
NPAD = 1008
OUTW = 128
NBUF = 4
NBKT = 32
CAND_CAP = 160
CAND_BUF = 1024
NC = 2
NS = 16
NW = NC * NS
ROWS_PER_W = B // NW
TC_ROWS = 256


def _iota16():
    return lax.broadcasted_iota(jnp.int32, (16,), 0)


class _Ops:
    def __init__(self, vsort1, rev, minimum, maximum, where, le):
        self.vsort1_f = vsort1
        self.rev_f = rev
        self.minimum = minimum
        self.maximum = maximum
        self.where = where
        self.le = le

    def vsort1(self, pair):
        if pair is None:
            return None
        return self.vsort1_f(pair)

    def rev(self, pair):
        if pair is None:
            return None
        return (self.rev_f(pair[0]), self.rev_f(pair[1]))

    def minmax(self, a, b):
        if a is None and b is None:
            return None, None
        if b is None:
            return a, None
        if a is None:
            return b, None
        ka, pa = a
        kb, pb = b
        m = self.le(ka, kb)
        lo = (self.minimum(ka, kb), self.where(m, pa, pb))
        hi = (self.maximum(ka, kb), self.where(m, pb, pa))
        return lo, hi

    def min_only(self, a, b):
        if b is None:
            return a
        if a is None:
            return b
        ka, pa = a
        kb, pb = b
        m = self.le(ka, kb)
        return (self.minimum(ka, kb), self.where(m, pa, pb))


def _bitonic(Z, ops, keep=None):
    n = len(Z)
    if keep is not None and keep <= 0:
        return [None] * n
    if n == 1:
        return [ops.vsort1(Z[0])]
    half = n // 2
    prune_hi = keep is not None and keep <= half
    lo, hi = [], []
    for i in range(half):
        if prune_hi:
            lo.append(ops.min_only(Z[i], Z[i + half]))
            hi.append(None)
        else:
            l, h = ops.minmax(Z[i], Z[i + half])
            lo.append(l)
            hi.append(h)
    khi = None if keep is None else keep - half
    return _bitonic(lo, ops, keep) + _bitonic(hi, ops, khi)


def _merge(A, Bl, ops, keep=None):
    n = 1
    while n < max(len(A), len(Bl)):
        n *= 2
    A = A + [None] * (n - len(A))
    Bl = Bl + [None] * (n - len(Bl))
    Z = A + [ops.rev(x) for x in reversed(Bl)]
    return _bitonic(Z, ops, keep)


def _sort_network(pairs, ops, keep=None):
    if len(pairs) == 1:
        return [ops.vsort1(pairs[0])]
    h = len(pairs) // 2
    return _merge(_sort_network(pairs[:h], ops),
                  _sort_network(pairs[h:], ops), ops, keep)


def _jnp_ops():
    def vsort1(pair):
        k, p = pair
        ks, ps = plsc.sort_key_val(k, p)
        return (ks, ps)
    return _Ops(vsort1, lambda v: lax.rev(v, (0,)), jnp.minimum,
                jnp.maximum, jnp.where, lambda a, b: a <= b)


def _topk_kernel(dist_hbm, theta_hbm, sd_hbm, th_hbm, ix_hbm, *scr):
    dbufs = scr[0:NBUF]
    tbufs = scr[NBUF:2 * NBUF]
    sdbufs = scr[2 * NBUF:3 * NBUF]
    thbufs = scr[3 * NBUF:4 * NBUF]
    ixbufs = scr[4 * NBUF:5 * NBUF]
    cand = scr[5 * NBUF]
    hist = scr[5 * NBUF + 1]
    sem_in = scr[5 * NBUF + 2:5 * NBUF + 2 + NBUF]
    sem_out = scr[5 * NBUF + 2 + NBUF:5 * NBUF + 2 + 2 * NBUF]

    wid = lax.axis_index("s") * NC + lax.axis_index("c")
    base = wid * ROWS_PER_W
    iota = _iota16()
    ones16 = jnp.full((16,), 1, jnp.int32)
    infv = jnp.full((16,), jnp.inf, jnp.float32)
    zf = jnp.zeros((16,), jnp.float32)
    zi = jnp.zeros((16,), jnp.int32)

    def in_copies(b, r):
        return (pltpu.make_async_copy(dist_hbm.at[base + r],
                                      dbufs[b].at[pl.ds(0, N)], sem_in[b]),
                pltpu.make_async_copy(theta_hbm.at[base + r],
                                      tbufs[b], sem_in[b]))

    def out_copies(b, r):
        return (pltpu.make_async_copy(sdbufs[b], sd_hbm.at[base + r], sem_out[b]),
                pltpu.make_async_copy(thbufs[b], th_hbm.at[base + r], sem_out[b]),
                pltpu.make_async_copy(ixbufs[b], ix_hbm.at[base + r], sem_out[b]))

    for b in range(NBUF):
        dbufs[b][pl.ds(992, 16)] = infv
        sdbufs[b][pl.ds(112, 16)] = zf
        thbufs[b][pl.ds(112, 16)] = zf
        ixbufs[b][pl.ds(112, 16)] = zi
        for c in in_copies(b, b):
            c.start()

    def process_row(b, tg):
        dbuf, tbuf = dbufs[b], tbufs[b]
        sdbuf, thbuf, ixbuf = sdbufs[b], thbufs[b], ixbufs[b]

        def prefill():
            for j in range(CAND_CAP // 16):
                cand[pl.ds(j * 16, 16)] = jnp.full((16,), NPAD - 1, jnp.int32)

        def compact(thresh):
            def body(i, off_v):
                for u in range(3):
                    c = i * 3 + u
                    v = dbuf[pl.ds(c * 16, 16)]
                    sel = v < thresh
                    cnt = jnp.where(sel, 1, 0)
                    pos = jnp.cumsum(cnt) - 1 + off_v
                    plsc.store_scatter(cand, [pos], c * 16 + iota, mask=sel)
                    off_v = off_v + plsc.all_reduce_population_count(sel)
                return off_v
            return lax.fori_loop(0, 21, body, jnp.zeros((16,), jnp.int32))

        prefill()
        m = jnp.max(compact(tg))
        ok = jnp.logical_and(m >= K, m <= CAND_CAP)

        def fast_fn(_):
            return m

        def exact_fn(_):
            for j in range(NBKT):
                hist[pl.ds(j * 16, 16)] = zi

            def hist_body(c, _):
                v = dbuf[pl.ds(c * 16, 16)]
                bkt = (v * float(NBKT)).astype(jnp.int32)
                addr = bkt * 16 + iota
                lanemask = (c * 16 + iota) < N
                plsc.addupdate_scatter(hist, [addr], ones16, mask=lanemask)
                return 0
            lax.fori_loop(0, 63, hist_body, 0)

            def scan_body(j, carry):
                cum, bstar = carry
                h = hist[pl.ds(j * 16, 16)]
                cum2 = cum + jnp.sum(h)
                hit = jnp.logical_and(bstar < 0, cum2 >= K)
                bstar = jnp.where(hit, j, bstar)
                return cum2, bstar
            _, bstar = lax.fori_loop(0, NBKT, scan_body,
                                     (jnp.int32(0), jnp.int32(-1)))
            bnd = (bstar + 1).astype(jnp.float32) * (1.0 / NBKT)
            prefill()
            return jnp.max(compact(bnd))

        m_f = lax.cond(ok, fast_fn, exact_fn, 0)

        ops = _jnp_ops()
        pairs = []
        for j in range(CAND_CAP // 16):
            iv = cand[pl.ds(j * 16, 16)]
            kv = plsc.load_gather(dbuf, [iv])
            pairs.append((kv, iv))
        pairs += [None] * (16 - len(pairs))
        snet = _sort_network(pairs, ops, keep=7)

        shift_iota = jnp.minimum(iota + 1, 15)
        tie = jnp.zeros((16,), jnp.bool_)
        prev_max = None
        for j in range(7):
            kj = snet[j][0]
            sh = jnp.take(kj, shift_iota)
            eq = jnp.logical_and(kj == sh, kj < jnp.inf)
            tie = jnp.logical_or(tie, jnp.logical_and(eq, iota < 15))
            if prev_max is not None:
                beq = jnp.logical_and(prev_max == jnp.min(kj),
                                      prev_max < jnp.inf)
                tie = jnp.logical_or(tie, jnp.full((16,), beq))
            prev_max = jnp.max(kj)
        n_tie = plsc.all_reduce_population_count(tie)
        bad = jnp.logical_or(jnp.logical_or(m_f > CAND_CAP, m_f < K),
                             jnp.sum(n_tie) > 0)

        for j in range(7):
            sdbuf[pl.ds(j * 16, 16)] = snet[j][0]
            ixbuf[pl.ds(j * 16, 16)] = snet[j][1]

        @pl.when(bad)
        def _fallback():
            def sel_body(k, _):
                def min_body(c, acc):
                    return jnp.minimum(acc, dbuf[pl.ds(c * 16, 16)])
                macc = lax.fori_loop(0, 63, min_body, infv)
                mn = jnp.min(macc)

                def pos_body(c, acc):
                    v = dbuf[pl.ds(c * 16, 16)]
                    cnd = jnp.where(v == mn, c * 16 + iota, NPAD)
                    return jnp.minimum(acc, cnd)
                pacc = lax.fori_loop(0, 63, pos_body,
                                     jnp.full((16,), NPAD, jnp.int32))
                p = jnp.min(pacc)
                lane0 = iota == 0
                kvec = jnp.full((16,), k, jnp.int32)
                plsc.store_scatter(sdbuf, [kvec], jnp.full((16,), mn), mask=lane0)
                plsc.store_scatter(ixbuf, [kvec], jnp.full((16,), p, jnp.int32),
                                   mask=lane0)
                plsc.store_scatter(dbuf, [jnp.full((16,), p, jnp.int32)], infv,
                                   mask=lane0)
                return 0
            lax.fori_loop(0, K, sel_body, 0)

        ix6 = jnp.where(iota < 4, ixbuf[pl.ds(96, 16)], 0)
        ixbuf[pl.ds(96, 16)] = ix6
        sd6 = jnp.where(iota < 4, sdbuf[pl.ds(96, 16)], 0.0)
        mx = jnp.max(sd6)
        for j in range(7):
            iv = ix6 if j == 6 else ixbuf[pl.ds(j * 16, 16)]
            tv = plsc.load_gather(tbuf, [iv])
            if j == 6:
                tv = jnp.where(iota < 4, tv, 0.0)
            thbuf[pl.ds(j * 16, 16)] = tv
            sv = sd6 if j == 6 else sdbuf[pl.ds(j * 16, 16)]
            sdbuf[pl.ds(j * 16, 16)] = sv / mx
        return mx * 1.25

    def loop_body(i, tg):
        r0 = i * NBUF
        for b in range(NBUF):
            r = r0 + b
            for c in in_copies(b, r):
                c.wait()

            @pl.when(r >= NBUF)
            def _wait_out():
                for c in out_copies(b, r - NBUF):
                    c.wait()

            tg = process_row(b, tg)
            for c in out_copies(b, r):
                c.start()

            @pl.when(r + NBUF < ROWS_PER_W)
            def _prefetch():
                for c in in_copies(b, r + NBUF):
                    c.start()
        return tg

    lax.fori_loop(0, ROWS_PER_W // NBUF, loop_body, jnp.float32(2.0))
    for b in range(NBUF):
        for c in out_copies(b, ROWS_PER_W - NBUF + b):
            c.wait()


def _make_topk():
    mesh = plsc.VectorSubcoreMesh(core_axis_name="c", subcore_axis_name="s")
    scratch = ([pltpu.VMEM((NPAD,), jnp.float32)] * NBUF +
               [pltpu.VMEM((N,), jnp.float32)] * NBUF +
               [pltpu.VMEM((OUTW,), jnp.float32)] * NBUF +
               [pltpu.VMEM((OUTW,), jnp.float32)] * NBUF +
               [pltpu.VMEM((OUTW,), jnp.int32)] * NBUF +
               [pltpu.VMEM((CAND_BUF,), jnp.int32),
                pltpu.VMEM((NBKT * 16,), jnp.int32)] +
               [pltpu.SemaphoreType.DMA] * (2 * NBUF))
    return pl.kernel(
        _topk_kernel,
        mesh=mesh,
        out_type=[jax.ShapeDtypeStruct((B, OUTW), jnp.float32),
                  jax.ShapeDtypeStruct((B, OUTW), jnp.float32),
                  jax.ShapeDtypeStruct((B, OUTW), jnp.int32)],
        scratch_types=scratch,
        compiler_params=pltpu.CompilerParams(needs_layout_passes=False, use_tc_tiling_on_sc=False),
    )


def _mlp_kernel(sd_ref, th_ref, ins_ref, w1a_ref, w1b_ref, w1c_ref, b1_ref,
                w2_ref, b2_ref, out_ref):
    sd = sd_ref[...]
    edge = jnp.dot(sd, w1a_ref[...], preferred_element_type=jnp.float32)
    edge += jnp.dot(th_ref[...], w1b_ref[...], preferred_element_type=jnp.float32)
    edge += jnp.dot(ins_ref[...], w1c_ref[...], preferred_element_type=jnp.float32)
    edge += b1_ref[...]
    out = jnp.dot(edge, w2_ref[...], preferred_element_type=jnp.float32)
    out_ref[...] = out + b2_ref[...] - sd


def _mlp(sd, th, ins2, w1a, w1b, w1c, b1, w2p, b2p):
    grid = (B // TC_ROWS,)
    return pl.pallas_call(
        _mlp_kernel,
        grid=grid,
        in_specs=[
            pl.BlockSpec((TC_ROWS, OUTW), lambda i: (i, 0)),
            pl.BlockSpec((TC_ROWS, OUTW), lambda i: (i, 0)),
            pl.BlockSpec((TC_ROWS, 2), lambda i: (i, 0)),
            pl.BlockSpec((OUTW, EMB), lambda i: (0, 0)),
            pl.BlockSpec((OUTW, EMB), lambda i: (0, 0)),
            pl.BlockSpec((2, EMB), lambda i: (0, 0)),
            pl.BlockSpec((1, EMB), lambda i: (0, 0)),
            pl.BlockSpec((EMB, OUTW), lambda i: (0, 0)),
            pl.BlockSpec((1, OUTW), lambda i: (0, 0)),
        ],
        out_specs=pl.BlockSpec((TC_ROWS, OUTW), lambda i: (i, 0)),
        out_shape=jax.ShapeDtypeStruct((B, OUTW), jnp.float32),
    )(sd, th, ins2, w1a, w1b, w1c, b1, w2p, b2p)


def _scatter_kernel(val_hbm, ix_hbm, om_hbm, *scr):
    rbufs = scr[0:NBUF]
    vbufs = scr[NBUF:2 * NBUF]
    ibufs = scr[2 * NBUF:3 * NBUF]
    sem_in = scr[3 * NBUF:4 * NBUF]
    sem_out = scr[4 * NBUF:5 * NBUF]

    wid = lax.axis_index("s") * NC + lax.axis_index("c")
    base = wid * ROWS_PER_W
    iota = _iota16()
    pen = jnp.full((16,), PENALTY, jnp.float32)

    def in_copies(b, r):
        return (pltpu.make_async_copy(val_hbm.at[base + r], vbufs[b], sem_in[b]),
                pltpu.make_async_copy(ix_hbm.at[base + r], ibufs[b], sem_in[b]))

    def out_copy(b, r):
        return pltpu.make_async_copy(rbufs[b].at[pl.ds(0, N)],
                                     om_hbm.at[base + r], sem_out[b])

    for b in range(NBUF):
        for c in in_copies(b, b):
            c.start()

    def loop_body(i, _):
        r0 = i * NBUF
        for b in range(NBUF):
            r = r0 + b
            for c in in_copies(b, r):
                c.wait()

            @pl.when(r >= NBUF)
            def _wait_out():
                out_copy(b, r - NBUF).wait()

            for j in range(63):
                rbufs[b][pl.ds(j * 16, 16)] = pen
            for j in range(7):
                iv = ibufs[b][pl.ds(j * 16, 16)]
                vv = vbufs[b][pl.ds(j * 16, 16)]
                mask = (j * 16 + iota) < K
                plsc.store_scatter(rbufs[b], [iv], vv, mask=mask)
            out_copy(b, r).start()

            @pl.when(r + NBUF < ROWS_PER_W)
            def _prefetch():
                for c in in_copies(b, r + NBUF):
                    c.start()
        return 0

    lax.fori_loop(0, ROWS_PER_W // NBUF, loop_body, 0)
    for b in range(NBUF):
        out_copy(b, ROWS_PER_W - NBUF + b).wait()


def _make_scatter():
    mesh = plsc.VectorSubcoreMesh(core_axis_name="c", subcore_axis_name="s")
    scratch = ([pltpu.VMEM((NPAD,), jnp.float32)] * NBUF +
               [pltpu.VMEM((OUTW,), jnp.float32)] * NBUF +
               [pltpu.VMEM((OUTW,), jnp.int32)] * NBUF +
               [pltpu.SemaphoreType.DMA] * (2 * NBUF))
    return pl.kernel(
        _scatter_kernel,
        mesh=mesh,
        out_type=jax.ShapeDtypeStruct((B, N), jnp.float32),
        scratch_types=scratch,
        compiler_params=pltpu.CompilerParams(needs_layout_passes=False, use_tc_tiling_on_sc=False),
    )


def kernel(theta, dist, ins_feature, W1, b1, W2, b2):
    sd, th, ix = _make_topk()(dist, theta)

    ins2 = jnp.concatenate([ins_feature[0], ins_feature[1]], axis=1)
    w1a = jnp.pad(W1[0:K], ((0, OUTW - K), (0, 0)))
    w1b = jnp.pad(W1[K:2 * K], ((0, OUTW - K), (0, 0)))
    w1c = W1[2 * K:2 * K + 2]
    w2p = jnp.pad(W2, ((0, 0), (0, OUTW - K)))
    b2p = jnp.pad(b2, (0, OUTW - K)).reshape(1, OUTW)
    b1r = b1.reshape(1, EMB)

    out = _mlp(sd, th, ins2, w1a, w1b, w1c, b1r, w2p, b2p)
    return _make_scatter()(out, ix)
